# baseline (speedup 1.0000x reference)
import functools

import jax
import jax.numpy as jnp
from jax import lax
from jax.experimental import pallas as pl
from jax.experimental.pallas import tpu as pltpu
from jax.experimental.pallas import tpu_sc as plsc

N_ECAL = 8000
N_ES = 2000
N = N_ECAL + N_ES
IN_DIM = 5
H = 64
K = 16
B = 16
R = 400
CB = 512
NPAD = 10752
NCHUNK = N // R
BIGI = 2 ** 30
INF = float("inf")
NEGINF = float("-inf")


def _expm1(v):
    p = v * (1.0 + v * (0.5 + v * (1.0 / 6.0 + v * (1.0 / 24.0 + v * (
        1.0 / 120.0 + v * (1.0 / 720.0 + v * (1.0 / 5040.0)))))))
    return jnp.where(v < -0.35, jnp.exp(v) - 1.0, p)


def _elu(v):
    return jnp.where(v > 0, v, _expm1(jnp.minimum(v, 0.0)))


def _elu_fast(v):
    return jnp.where(v > 0, v, jnp.exp(jnp.minimum(v, 0.0)) - 1.0)


def _dot(a, b):
    return jnp.dot(a, b, preferred_element_type=jnp.float32)


def _embed_body(xin_ref, we_ref, be_ref, ws_ref, bs_ref, out_ref):
    i = pl.program_id(0)
    use_ecal = i < (N_ECAL // R)
    wsel = jnp.where(use_ecal, we_ref[...], ws_ref[...])
    bsel = jnp.where(use_ecal, be_ref[...], bs_ref[...])
    out_ref[...] = _elu(_dot(xin_ref[...], wsel) + bsel)


def _prep_body(x_ref, sq_ref):
    xb = x_ref[...]
    sq_ref[...] = jnp.sum(xb * xb, axis=1, keepdims=True)


def _knn_body(cinfo_ref, x_ref, xpad_ref, sqc_ref, sqr_ref, brow_ref,
              bcol_ref, out_ref):
    i = pl.program_id(0)
    cbase = cinfo_ref[0, i]
    nv = cinfo_ref[1, i]
    xr = x_ref[...]
    sqr = sqr_ref[...]
    br = brow_ref[...]
    kiota = jax.lax.broadcasted_iota(jnp.int32, (R, K), 1)

    def tk_body(c, carry):
        vals, idxs = carry
        colbase = pl.multiple_of(cbase + c * CB, 128)
        xc = xpad_ref[pl.ds(colbase, CB), :]
        sc = sqc_ref[:, pl.ds(colbase, CB)]
        bc = bcol_ref[:, pl.ds(colbase, CB)]
        d = (sqr - 2.0 * jax.lax.dot_general(
            xr, xc, (((1,), (1,)), ((), ())),
            preferred_element_type=jnp.float32)) + sc
        d = jnp.where(br == bc, d, INF)
        gidx = colbase + jax.lax.broadcasted_iota(jnp.int32, (R, CB), 1)
        allv = jnp.concatenate([vals, d], axis=1)
        alli = jnp.concatenate([idxs, gidx], axis=1)
        for k in range(K):
            m = jnp.min(allv, axis=1, keepdims=True)
            j = jnp.min(jnp.where(allv == m, alli, BIGI), axis=1,
                        keepdims=True)
            vals = jnp.where(kiota == k, m, vals)
            idxs = jnp.where(kiota == k, j, idxs)
            allv = jnp.where(alli == j, INF, allv)
        return vals, idxs

    vals0 = jnp.full((R, K), INF, jnp.float32)
    idxs0 = jnp.full((R, K), BIGI, jnp.int32)
    _, idxs = jax.lax.fori_loop(0, nv, tk_body, (vals0, idxs0))
    out_ref[...] = jnp.clip(idxs, 0, N - 1)


SC_NC = 2
SC_NS = 16
SC_NW = SC_NC * SC_NS
SPLITS = ((12, 800), (13, 520))


def _make_sc_gather(n_idx, gb):
    bw = n_idx // SC_NW

    def body(x_hbm, idx_hbm, out_hbm, idx_v, rows_v, sem):
        wid = lax.axis_index("s") * SC_NC + lax.axis_index("c")
        base = wid * bw

        def step(j, carry):
            off = base + j * gb
            pltpu.sync_copy(idx_hbm.at[pl.ds(off, gb)], idx_v)
            pltpu.async_copy(x_hbm.at[idx_v], rows_v, sem).wait()
            pltpu.sync_copy(rows_v, out_hbm.at[pl.ds(off, gb)])
            return carry

        lax.fori_loop(0, bw // gb, step, 0)

    return pl.kernel(
        body,
        out_type=jax.ShapeDtypeStruct((n_idx, H), jnp.float32),
        mesh=plsc.VectorSubcoreMesh(core_axis_name="c",
                                    subcore_axis_name="s"),
        scratch_types=[
            pltpu.VMEM((gb,), jnp.int32),
            pltpu.VMEM((gb, H), jnp.float32),
            pltpu.SemaphoreType.DMA,
        ],
        compiler_params=pltpu.CompilerParams(use_tc_tiling_on_sc=False),
    )


def _mlp_body(x_ref, xg_ref, w1_ref, b1_ref, w2_ref, b2_ref, out_ref,
              *, elu):
    xr = x_ref[...]
    xg = xg_ref[...]
    xi = jnp.broadcast_to(xr[:, None, :], (R, K, H)).reshape(R * K, H)
    feat = jnp.concatenate([xi, xg - xi], axis=1)
    h1 = elu(_dot(feat, w1_ref[...]) + b1_ref[...])
    m = elu(_dot(h1, w2_ref[...]) + b2_ref[...])
    out_ref[...] = jnp.sum(m.reshape(R, K, H), axis=1)


def _pool_body(x_ref, brow_ref, wo1_ref, bo1_ref, wo2_ref, bo2_ref,
               wo3_ref, bo3_ref, out_ref):
    xv = x_ref[...]
    bv = brow_ref[...]
    rows = [jnp.max(jnp.where(bv == b, xv, NEGINF), axis=0, keepdims=True)
            for b in range(B)]
    pooled = jnp.concatenate(rows, axis=0)
    o = _elu_fast(_dot(pooled, wo1_ref[...]) + bo1_ref[...])
    o = _elu_fast(_dot(o, wo2_ref[...]) + bo2_ref[...])
    out_ref[...] = _dot(o, wo3_ref[...]) + bo3_ref[...]


def kernel(xECAL, xES, batch, W_in_ecal, b_in_ecal, W_in_es, b_in_es,
           W1_0, b1_0, W2_0, b2_0, W1_1, b1_1, W2_1, b2_1,
           Wo1, bo1, Wo2, bo2, Wo3, bo3):
    xin = jnp.concatenate([xECAL, xES], axis=0)
    batch = batch.astype(jnp.int32)

    x = pl.pallas_call(
        _embed_body,
        grid=(NCHUNK,),
        in_specs=[
            pl.BlockSpec((R, IN_DIM), lambda i: (i, 0)),
            pl.BlockSpec((IN_DIM, H), lambda i: (0, 0)),
            pl.BlockSpec((1, H), lambda i: (0, 0)),
            pl.BlockSpec((IN_DIM, H), lambda i: (0, 0)),
            pl.BlockSpec((1, H), lambda i: (0, 0)),
        ],
        out_specs=pl.BlockSpec((R, H), lambda i: (i, 0)),
        out_shape=jax.ShapeDtypeStruct((N, H), jnp.float32),
    )(xin, W_in_ecal, b_in_ecal.reshape(1, H), W_in_es, b_in_es.reshape(1, H))

    offs = jnp.searchsorted(batch, jnp.arange(B + 1, dtype=jnp.int32)).astype(jnp.int32)
    row_starts = jnp.arange(NCHUNK, dtype=jnp.int32) * R
    b_lo = batch[row_starts]
    b_hi = batch[row_starts + R - 1]
    cbase = (offs[b_lo] // 128) * 128
    nvisit = (offs[b_hi + 1] - cbase + CB - 1) // CB
    cinfo = jnp.stack([cbase, nvisit], axis=0)

    batch_col = jnp.pad(batch, (0, NPAD - N), constant_values=-1).reshape(1, NPAD)
    batch_row = batch.reshape(N, 1)

    for lyr, (W1, b1, W2, b2) in enumerate(((W1_0, b1_0, W2_0, b2_0),
                                            (W1_1, b1_1, W2_1, b2_1))):
        sq = pl.pallas_call(
            _prep_body,
            grid=(NCHUNK,),
            in_specs=[pl.BlockSpec((R, H), lambda i: (i, 0))],
            out_specs=pl.BlockSpec((R, 1), lambda i: (i, 0)),
            out_shape=jax.ShapeDtypeStruct((N, 1), jnp.float32),
        )(x)

        xpad = jnp.pad(x, ((0, NPAD - N), (0, 0)))
        sq_col = jnp.pad(sq.reshape(1, N), ((0, 0), (0, NPAD - N)))

        grid_spec = pltpu.PrefetchScalarGridSpec(
            num_scalar_prefetch=1,
            grid=(NCHUNK,),
            in_specs=[
                pl.BlockSpec((R, H), lambda i, s: (i, 0)),
                pl.BlockSpec((NPAD, H), lambda i, s: (0, 0)),
                pl.BlockSpec((1, NPAD), lambda i, s: (0, 0)),
                pl.BlockSpec((R, 1), lambda i, s: (i, 0)),
                pl.BlockSpec((R, 1), lambda i, s: (i, 0)),
                pl.BlockSpec((1, NPAD), lambda i, s: (0, 0)),
            ],
            out_specs=pl.BlockSpec((R, K), lambda i, s: (i, 0)),
        )
        idx = pl.pallas_call(
            _knn_body,
            grid_spec=grid_spec,
            out_shape=jax.ShapeDtypeStruct((N, K), jnp.int32),
        )(cinfo, x, xpad, sq_col, sq, batch_row, batch_col)

        idxf = idx.reshape(N * K)
        elu = _elu if lyr == 0 else _elu_fast
        row0 = 0
        gathered = []
        for (nch, gb) in SPLITS:
            nrow = nch * R
            gathered.append(_make_sc_gather(nrow * K, gb)(
                x, lax.dynamic_slice_in_dim(idxf, row0 * K, nrow * K)))
            row0 += nrow
        row0 = 0
        pieces = []
        for (nch, gb), xg in zip(SPLITS, gathered):
            nrow = nch * R
            xpiece = lax.dynamic_slice_in_dim(x, row0, nrow)
            pieces.append(pl.pallas_call(
                functools.partial(_mlp_body, elu=elu),
                grid=(nch,),
                in_specs=[
                    pl.BlockSpec((R, H), lambda i: (i, 0)),
                    pl.BlockSpec((R * K, H), lambda i: (i, 0)),
                    pl.BlockSpec((2 * H, H), lambda i: (0, 0)),
                    pl.BlockSpec((1, H), lambda i: (0, 0)),
                    pl.BlockSpec((H, H), lambda i: (0, 0)),
                    pl.BlockSpec((1, H), lambda i: (0, 0)),
                ],
                out_specs=pl.BlockSpec((R, H), lambda i: (i, 0)),
                out_shape=jax.ShapeDtypeStruct((nrow, H), jnp.float32),
            )(xpiece, xg, W1, b1.reshape(1, H), W2, b2.reshape(1, H)))
            row0 += nrow
        x = jnp.concatenate(pieces, axis=0)

    out = pl.pallas_call(
        _pool_body,
        in_specs=[
            pl.BlockSpec((N, H), lambda: (0, 0)),
            pl.BlockSpec((N, 1), lambda: (0, 0)),
            pl.BlockSpec((H, H), lambda: (0, 0)),
            pl.BlockSpec((1, H), lambda: (0, 0)),
            pl.BlockSpec((H, H // 2), lambda: (0, 0)),
            pl.BlockSpec((1, H // 2), lambda: (0, 0)),
            pl.BlockSpec((H // 2, 1), lambda: (0, 0)),
            pl.BlockSpec((1, 1), lambda: (0, 0)),
        ],
        out_specs=pl.BlockSpec((B, 1), lambda: (0, 0)),
        out_shape=jax.ShapeDtypeStruct((B, 1), jnp.float32),
    )(x, batch_row, Wo1, bo1.reshape(1, H), Wo2, bo2.reshape(1, H // 2),
      Wo3, bo3.reshape(1, 1))
    return out


# speedup vs baseline: 1.0111x; 1.0111x over previous
import functools

import jax
import jax.numpy as jnp
from jax import lax
from jax.experimental import pallas as pl
from jax.experimental.pallas import tpu as pltpu
from jax.experimental.pallas import tpu_sc as plsc

N_ECAL = 8000
N_ES = 2000
N = N_ECAL + N_ES
IN_DIM = 5
H = 64
K = 16
B = 16
R = 400
CB = 512
NPAD = 10752
NCHUNK = N // R
BIGI = 2 ** 30
INF = float("inf")
NEGINF = float("-inf")


def _expm1(v):
    p = v * (1.0 + v * (0.5 + v * (1.0 / 6.0 + v * (1.0 / 24.0 + v * (
        1.0 / 120.0 + v * (1.0 / 720.0 + v * (1.0 / 5040.0)))))))
    return jnp.where(v < -0.35, jnp.exp(v) - 1.0, p)


def _elu(v):
    return jnp.where(v > 0, v, _expm1(jnp.minimum(v, 0.0)))


def _elu_fast(v):
    return jnp.where(v > 0, v, jnp.exp(jnp.minimum(v, 0.0)) - 1.0)


def _dot(a, b):
    return jnp.dot(a, b, preferred_element_type=jnp.float32)


def _embed_body(xin_ref, we_ref, be_ref, ws_ref, bs_ref, out_ref):
    i = pl.program_id(0)
    use_ecal = i < (N_ECAL // R)
    wsel = jnp.where(use_ecal, we_ref[...], ws_ref[...])
    bsel = jnp.where(use_ecal, be_ref[...], bs_ref[...])
    out_ref[...] = _elu(_dot(xin_ref[...], wsel) + bsel)


def _prep_body(x_ref, sq_ref):
    xb = x_ref[...]
    sq_ref[...] = jnp.sum(xb * xb, axis=1, keepdims=True)


def _knn_body(cinfo_ref, x_ref, xpad_ref, sqc_ref, sqr_ref, brow_ref,
              bcol_ref, out_ref):
    i = pl.program_id(0)
    cbase = cinfo_ref[0, i]
    nv = cinfo_ref[1, i]
    xr = x_ref[...]
    sqr = sqr_ref[...]
    br = brow_ref[...]
    kiota = jax.lax.broadcasted_iota(jnp.int32, (R, K), 1)

    def tk_body(c, carry):
        vals, idxs = carry
        colbase = pl.multiple_of(cbase + c * CB, 128)
        xc = xpad_ref[pl.ds(colbase, CB), :]
        sc = sqc_ref[:, pl.ds(colbase, CB)]
        bc = bcol_ref[:, pl.ds(colbase, CB)]
        d = (sqr - 2.0 * jax.lax.dot_general(
            xr, xc, (((1,), (1,)), ((), ())),
            preferred_element_type=jnp.float32)) + sc
        d = jnp.where(br == bc, d, INF)
        gidx = colbase + jax.lax.broadcasted_iota(jnp.int32, (R, CB), 1)
        allv = jnp.concatenate([vals, d], axis=1)
        alli = jnp.concatenate([idxs, gidx], axis=1)
        for k in range(K):
            m = jnp.min(allv, axis=1, keepdims=True)
            j = jnp.min(jnp.where(allv == m, alli, BIGI), axis=1,
                        keepdims=True)
            vals = jnp.where(kiota == k, m, vals)
            idxs = jnp.where(kiota == k, j, idxs)
            allv = jnp.where(alli == j, INF, allv)
        return vals, idxs

    vals0 = jnp.full((R, K), INF, jnp.float32)
    idxs0 = jnp.full((R, K), BIGI, jnp.int32)
    _, idxs = jax.lax.fori_loop(0, nv, tk_body, (vals0, idxs0))
    out_ref[...] = jnp.clip(idxs, 0, N - 1)


SC_NC = 2
SC_NS = 16
SC_NW = SC_NC * SC_NS


def _make_sc_gather(n_idx, gb):
    bw = n_idx // SC_NW

    def body(x_hbm, idx_hbm, out_hbm, idx_v, rows_v, sem):
        wid = lax.axis_index("s") * SC_NC + lax.axis_index("c")
        base = wid * bw

        def step(j, carry):
            off = base + j * gb
            pltpu.sync_copy(idx_hbm.at[pl.ds(off, gb)], idx_v)
            pltpu.async_copy(x_hbm.at[idx_v], rows_v, sem).wait()
            pltpu.sync_copy(rows_v, out_hbm.at[pl.ds(off, gb)])
            return carry

        lax.fori_loop(0, bw // gb, step, 0)

    return pl.kernel(
        body,
        out_type=jax.ShapeDtypeStruct((n_idx, H), jnp.float32),
        mesh=plsc.VectorSubcoreMesh(core_axis_name="c",
                                    subcore_axis_name="s"),
        scratch_types=[
            pltpu.VMEM((gb,), jnp.int32),
            pltpu.VMEM((gb, H), jnp.float32),
            pltpu.SemaphoreType.DMA,
        ],
        compiler_params=pltpu.CompilerParams(use_tc_tiling_on_sc=False),
    )


def _mlp_body(x_ref, xg_ref, w1_ref, b1_ref, w2_ref, b2_ref, out_ref,
              *, elu):
    xr = x_ref[...]
    xg = xg_ref[...]
    xi = jnp.broadcast_to(xr[:, None, :], (R, K, H)).reshape(R * K, H)
    feat = jnp.concatenate([xi, xg - xi], axis=1)
    h1 = elu(_dot(feat, w1_ref[...]) + b1_ref[...])
    m = elu(_dot(h1, w2_ref[...]) + b2_ref[...])
    out_ref[...] = jnp.sum(m.reshape(R, K, H), axis=1)


def _pool_body(x_ref, brow_ref, wo1_ref, bo1_ref, wo2_ref, bo2_ref,
               wo3_ref, bo3_ref, out_ref):
    xv = x_ref[...]
    bv = brow_ref[...]
    rows = [jnp.max(jnp.where(bv == b, xv, NEGINF), axis=0, keepdims=True)
            for b in range(B)]
    pooled = jnp.concatenate(rows, axis=0)
    o = _elu_fast(_dot(pooled, wo1_ref[...]) + bo1_ref[...])
    o = _elu_fast(_dot(o, wo2_ref[...]) + bo2_ref[...])
    out_ref[...] = _dot(o, wo3_ref[...]) + bo3_ref[...]


def kernel(xECAL, xES, batch, W_in_ecal, b_in_ecal, W_in_es, b_in_es,
           W1_0, b1_0, W2_0, b2_0, W1_1, b1_1, W2_1, b2_1,
           Wo1, bo1, Wo2, bo2, Wo3, bo3):
    xin = jnp.concatenate([xECAL, xES], axis=0)
    batch = batch.astype(jnp.int32)

    x = pl.pallas_call(
        _embed_body,
        grid=(NCHUNK,),
        in_specs=[
            pl.BlockSpec((R, IN_DIM), lambda i: (i, 0)),
            pl.BlockSpec((IN_DIM, H), lambda i: (0, 0)),
            pl.BlockSpec((1, H), lambda i: (0, 0)),
            pl.BlockSpec((IN_DIM, H), lambda i: (0, 0)),
            pl.BlockSpec((1, H), lambda i: (0, 0)),
        ],
        out_specs=pl.BlockSpec((R, H), lambda i: (i, 0)),
        out_shape=jax.ShapeDtypeStruct((N, H), jnp.float32),
    )(xin, W_in_ecal, b_in_ecal.reshape(1, H), W_in_es, b_in_es.reshape(1, H))

    offs = jnp.searchsorted(batch, jnp.arange(B + 1, dtype=jnp.int32)).astype(jnp.int32)
    row_starts = jnp.arange(NCHUNK, dtype=jnp.int32) * R
    b_lo = batch[row_starts]
    b_hi = batch[row_starts + R - 1]
    cbase = (offs[b_lo] // 128) * 128
    nvisit = (offs[b_hi + 1] - cbase + CB - 1) // CB
    cinfo = jnp.stack([cbase, nvisit], axis=0)

    batch_col = jnp.pad(batch, (0, NPAD - N), constant_values=-1).reshape(1, NPAD)
    batch_row = batch.reshape(N, 1)

    for lyr, (W1, b1, W2, b2) in enumerate(((W1_0, b1_0, W2_0, b2_0),
                                            (W1_1, b1_1, W2_1, b2_1))):
        sq = pl.pallas_call(
            _prep_body,
            grid=(NCHUNK,),
            in_specs=[pl.BlockSpec((R, H), lambda i: (i, 0))],
            out_specs=pl.BlockSpec((R, 1), lambda i: (i, 0)),
            out_shape=jax.ShapeDtypeStruct((N, 1), jnp.float32),
        )(x)

        xpad = jnp.pad(x, ((0, NPAD - N), (0, 0)))
        sq_col = jnp.pad(sq.reshape(1, N), ((0, 0), (0, NPAD - N)))

        grid_spec = pltpu.PrefetchScalarGridSpec(
            num_scalar_prefetch=1,
            grid=(NCHUNK,),
            in_specs=[
                pl.BlockSpec((R, H), lambda i, s: (i, 0)),
                pl.BlockSpec((NPAD, H), lambda i, s: (0, 0)),
                pl.BlockSpec((1, NPAD), lambda i, s: (0, 0)),
                pl.BlockSpec((R, 1), lambda i, s: (i, 0)),
                pl.BlockSpec((R, 1), lambda i, s: (i, 0)),
                pl.BlockSpec((1, NPAD), lambda i, s: (0, 0)),
            ],
            out_specs=pl.BlockSpec((R, K), lambda i, s: (i, 0)),
        )
        idx = pl.pallas_call(
            _knn_body,
            grid_spec=grid_spec,
            out_shape=jax.ShapeDtypeStruct((N, K), jnp.int32),
        )(cinfo, x, xpad, sq_col, sq, batch_row, batch_col)

        xg = _make_sc_gather(N * K, 1000)(x, idx.reshape(N * K))

        x = pl.pallas_call(
            functools.partial(_mlp_body,
                              elu=_elu if lyr == 0 else _elu_fast),
            grid=(NCHUNK,),
            in_specs=[
                pl.BlockSpec((R, H), lambda i: (i, 0)),
                pl.BlockSpec((R * K, H), lambda i: (i, 0)),
                pl.BlockSpec((2 * H, H), lambda i: (0, 0)),
                pl.BlockSpec((1, H), lambda i: (0, 0)),
                pl.BlockSpec((H, H), lambda i: (0, 0)),
                pl.BlockSpec((1, H), lambda i: (0, 0)),
            ],
            out_specs=pl.BlockSpec((R, H), lambda i: (i, 0)),
            out_shape=jax.ShapeDtypeStruct((N, H), jnp.float32),
        )(x, xg, W1, b1.reshape(1, H), W2, b2.reshape(1, H))

    out = pl.pallas_call(
        _pool_body,
        in_specs=[
            pl.BlockSpec((N, H), lambda: (0, 0)),
            pl.BlockSpec((N, 1), lambda: (0, 0)),
            pl.BlockSpec((H, H), lambda: (0, 0)),
            pl.BlockSpec((1, H), lambda: (0, 0)),
            pl.BlockSpec((H, H // 2), lambda: (0, 0)),
            pl.BlockSpec((1, H // 2), lambda: (0, 0)),
            pl.BlockSpec((H // 2, 1), lambda: (0, 0)),
            pl.BlockSpec((1, 1), lambda: (0, 0)),
        ],
        out_specs=pl.BlockSpec((B, 1), lambda: (0, 0)),
        out_shape=jax.ShapeDtypeStruct((B, 1), jnp.float32),
    )(x, batch_row, Wo1, bo1.reshape(1, H), Wo2, bo2.reshape(1, H // 2),
      Wo3, bo3.reshape(1, 1))
    return out
